# baseline (speedup 1.0000x reference)
import jax
import jax.numpy as jnp
from jax import lax
from jax.experimental import pallas as pl
from jax.experimental.pallas import tpu as pltpu
from jax.experimental.pallas import tpu_sc as plsc

N = 100000
HD = 32
TD = 64
B = 4
GAMMA = 12.0
PI = 3.141592653589793
EMB_RANGE = (GAMMA + 2.0) / HD
PHASE_SCALE = PI / EMB_RANGE

NW = 32
CHUNK = 640
NCHUNK = 5
PER_W = CHUNK * NCHUNK
NPAD = NW * PER_W
GROUPS = CHUNK // 16
CROWS = CHUNK // 2

NEG_MAGIC = (0x5F3759DF | 0x80000000) - (1 << 32)


def _sincos16(ph):
    t = ph * jnp.float32(1.0 / PI)
    half = jnp.where(t >= 0, jnp.float32(0.5), jnp.float32(-0.5))
    n = (t + half).astype(jnp.int32)
    y = ph - n.astype(jnp.float32) * jnp.float32(PI)
    y2 = y * y
    cy = jnp.float32(-1.0 / 3628800)
    for c in (1.0 / 40320, -1.0 / 720, 1.0 / 24, -0.5):
        cy = cy * y2 + jnp.float32(c)
    cy = cy * y2 + jnp.float32(1.0)
    sy = jnp.float32(1.0 / 362880)
    for c in (-1.0 / 5040, 1.0 / 120, -1.0 / 6):
        sy = sy * y2 + jnp.float32(c)
    sy = (sy * y2 + jnp.float32(1.0)) * y
    sgn = jnp.where((n & 1) == 0, jnp.float32(1.0), jnp.float32(-1.0))
    return sgn * sy, sgn * cy


def _body(hrow_hbm, rrow_hbm, meta_hbm, eemb_hbm, remb_hbm, out_hbm,
          idxh, idxr, metav, hbuf, rbuf, qrot, ridx, scores,
          ebuf0, ebuf1, semg, sem0, sem1):
    wid = lax.axis_index("s") * 2 + lax.axis_index("c")
    wbase = wid * PER_W
    it16 = lax.iota(jnp.int32, 16)
    gamma16 = jnp.full((16,), jnp.float32(GAMMA))
    colbase = (it16 & 1) << 6

    pltpu.sync_copy(hrow_hbm, idxh)
    pltpu.sync_copy(rrow_hbm, idxr)
    pltpu.sync_copy(meta_hbm, metav)
    pltpu.async_copy(eemb_hbm.at[idxh], hbuf, semg).wait()
    pltpu.async_copy(remb_hbm.at[idxr], rbuf, semg).wait()
    for b in range(B):
        bsp = jnp.full((16,), b, jnp.int32)
        hoff = plsc.load_gather(metav, [bsp + 4])
        roff = plsc.load_gather(metav, [bsp + 8])
        for j in range(2):
            lane = it16 + j * 16
            ph = plsc.load_gather(rbuf, [bsp, roff + lane])
            ph = ph * jnp.float32(PHASE_SCALE)
            sy, cy = _sincos16(ph)
            reh = plsc.load_gather(hbuf, [bsp, hoff + lane])
            imh = plsc.load_gather(hbuf, [bsp, hoff + lane + HD])
            sl = pl.ds(j * 16, 16)
            sli = pl.ds(HD + j * 16, 16)
            qrot[b + 1, sl] = reh * cy - imh * sy
            qrot[b + 1, sli] = reh * sy + imh * cy

    bufs = (ebuf0, ebuf1)
    sems = (sem0, sem1)

    def start_load(c, buf, sem):
        rbase = wbase + c * CHUNK
        src = pl.multiple_of(jnp.minimum(rbase, jnp.int32(N - CHUNK)) // 2, 8)
        pltpu.make_async_copy(eemb_hbm.at[pl.ds(src, CROWS)], buf, sem).start()

    def wait_load(buf, sem):
        pltpu.make_async_copy(eemb_hbm.at[pl.ds(0, CROWS)], buf, sem).wait()

    def compute_chunk(c, buf):
        rbase = wbase + c * CHUNK
        delta = rbase - 2 * (jnp.minimum(rbase, jnp.int32(N - CHUNK)) // 2)

        def init_g(g, _):
            ent = delta + g * 16 + it16
            ridx[g, :] = jnp.minimum(ent >> 1, jnp.int32(CROWS - 1))
            gs = g * 16
            for b in range(B):
                scores[b, pl.ds(gs, 16)] = gamma16
            return 0

        lax.fori_loop(0, GROUPS, init_g, 0)

        def k_body(k, _):
            kre = colbase + jnp.broadcast_to(k, (16,))
            kim = kre + HD
            ksp = jnp.broadcast_to(k, (16,))
            qv = []
            for b in range(B):
                bsp = jnp.full((16,), b + 1, jnp.int32)
                qv.append((plsc.load_gather(qrot, [bsp, ksp]),
                           plsc.load_gather(qrot, [bsp, ksp + HD])))

            @plsc.parallel_loop(0, GROUPS, unroll=4)
            def g_body(g):
                rows = ridx[g, :]
                e_re = plsc.load_gather(buf, [rows, kre])
                e_im = plsc.load_gather(buf, [rows, kim])
                gs = g * 16
                for b in range(B):
                    qre, qim = qv[b]
                    dre = qre - e_re
                    dim = qim - e_im
                    dsq = dre * dre + dim * dim
                    i = NEG_MAGIC - (plsc.bitcast(dsq, jnp.int32) >> 1)
                    y = plsc.bitcast(i, jnp.float32)
                    y = y * (jnp.float32(1.5)
                             - (jnp.float32(0.5) * dsq) * (y * y))
                    plsc.addupdate(scores.at[b, pl.ds(gs, 16)], dsq * y)

            return 0

        lax.fori_loop(0, HD, k_body, 0)
        col = pl.multiple_of(rbase, 128)
        pltpu.sync_copy(scores, out_hbm.at[:, pl.ds(col, CHUNK)])

    start_load(0, bufs[0], sems[0])
    for c in range(NCHUNK):
        if c + 1 < NCHUNK:
            start_load(c + 1, bufs[(c + 1) % 2], sems[(c + 1) % 2])
        wait_load(bufs[c % 2], sems[c % 2])
        compute_chunk(c, bufs[c % 2])


def kernel(all_h, all_r, eemb, remb):
    mesh = plsc.VectorSubcoreMesh(core_axis_name="c", subcore_axis_name="s")
    run = pl.kernel(
        _body,
        out_type=jax.ShapeDtypeStruct((B, NPAD), jnp.float32),
        scratch_types=[
            pltpu.VMEM((B,), jnp.int32),
            pltpu.VMEM((B,), jnp.int32),
            pltpu.VMEM((16,), jnp.int32),
            pltpu.VMEM((B, 128), jnp.float32),
            pltpu.VMEM((B, 128), jnp.float32),
            pltpu.VMEM((B + 1, TD), jnp.float32),
            pltpu.VMEM((GROUPS, 16), jnp.int32),
            pltpu.VMEM((B, CHUNK), jnp.float32),
            pltpu.VMEM((CROWS, 128), jnp.float32),
            pltpu.VMEM((CROWS, 128), jnp.float32),
            pltpu.SemaphoreType.DMA,
            pltpu.SemaphoreType.DMA,
            pltpu.SemaphoreType.DMA,
        ],
        mesh=mesh,
        compiler_params=pltpu.CompilerParams(needs_layout_passes=False),
    )
    all_h = all_h.astype(jnp.int32)
    all_r = all_r.astype(jnp.int32)
    zero4 = jnp.zeros((4,), jnp.int32)
    meta = jnp.concatenate([zero4, (all_h % 2) * TD, (all_r % 4) * HD, zero4])
    out = run(
        all_h // 2, all_r // 4, meta,
        eemb.reshape(N // 2, 128), remb.reshape(125, 128),
    )
    return out[:, :N]


# speedup vs baseline: 1.6441x; 1.6441x over previous
import jax
import jax.numpy as jnp
from jax import lax
from jax.experimental import pallas as pl
from jax.experimental.pallas import tpu as pltpu
from jax.experimental.pallas import tpu_sc as plsc

N = 100000
HD = 32
TD = 64
B = 4
GAMMA = 12.0
PI = 3.141592653589793
EMB_RANGE = (GAMMA + 2.0) / HD
PHASE_SCALE = PI / EMB_RANGE

NW = 32
CHUNK = 640
NCHUNK = 5
PER_W = CHUNK * NCHUNK
NPAD = NW * PER_W
GROUPS = CHUNK // 16
CROWS = CHUNK // 2
ESTRIDE = 67

NEG_MAGIC = (0x5F3759DF | 0x80000000) - (1 << 32)


def _sincos16(ph):
    t = ph * jnp.float32(1.0 / PI)
    half = jnp.where(t >= 0, jnp.float32(0.5), jnp.float32(-0.5))
    n = (t + half).astype(jnp.int32)
    y = ph - n.astype(jnp.float32) * jnp.float32(PI)
    y2 = y * y
    cy = jnp.float32(-1.0 / 3628800)
    for c in (1.0 / 40320, -1.0 / 720, 1.0 / 24, -0.5):
        cy = cy * y2 + jnp.float32(c)
    cy = cy * y2 + jnp.float32(1.0)
    sy = jnp.float32(1.0 / 362880)
    for c in (-1.0 / 5040, 1.0 / 120, -1.0 / 6):
        sy = sy * y2 + jnp.float32(c)
    sy = (sy * y2 + jnp.float32(1.0)) * y
    sgn = jnp.where((n & 1) == 0, jnp.float32(1.0), jnp.float32(-1.0))
    return sgn * sy, sgn * cy


def _body(hrow_hbm, rrow_hbm, meta_hbm, eemb_hbm, remb_hbm, out_hbm,
          idxh, idxr, metav, hbuf, rbuf, qrot, ridx, scores,
          stage, pbuf, semg, sem0):
    wid = lax.axis_index("s") * 2 + lax.axis_index("c")
    wbase = wid * PER_W
    it16 = lax.iota(jnp.int32, 16)
    gamma16 = jnp.full((16,), jnp.float32(GAMMA))

    pltpu.sync_copy(hrow_hbm, idxh)
    pltpu.sync_copy(rrow_hbm, idxr)
    pltpu.sync_copy(meta_hbm, metav)
    pltpu.async_copy(eemb_hbm.at[idxh], hbuf, semg).wait()
    pltpu.async_copy(remb_hbm.at[idxr], rbuf, semg).wait()
    for b in range(B):
        bsp = jnp.full((16,), b, jnp.int32)
        hoff = plsc.load_gather(metav, [bsp + 4])
        roff = plsc.load_gather(metav, [bsp + 8])
        for j in range(2):
            lane = it16 + j * 16
            ph = plsc.load_gather(rbuf, [bsp, roff + lane])
            ph = ph * jnp.float32(PHASE_SCALE)
            sy, cy = _sincos16(ph)
            reh = plsc.load_gather(hbuf, [bsp, hoff + lane])
            imh = plsc.load_gather(hbuf, [bsp, hoff + lane + HD])
            sl = pl.ds(j * 16, 16)
            sli = pl.ds(HD + j * 16, 16)
            qrot[b + 1, sl] = reh * cy - imh * sy
            qrot[b + 1, sli] = reh * sy + imh * cy


    def start_load(c):
        rbase = wbase + c * CHUNK
        src = pl.multiple_of(jnp.minimum(rbase, jnp.int32(N - CHUNK)) // 2, 8)
        pltpu.make_async_copy(eemb_hbm.at[pl.ds(src, CROWS)], stage, sem0).start()

    def wait_load():
        pltpu.make_async_copy(eemb_hbm.at[pl.ds(0, CROWS)], stage, sem0).wait()

    def relayout():
        @plsc.parallel_loop(0, CHUNK, unroll=2)
        def _(e):
            half = (e & 1) * TD
            row = e >> 1
            base = e * ESTRIDE
            for cb in range(4):
                v = stage[row, pl.ds(half + cb * 16, 16)]
                pbuf[pl.ds(base + cb * 16, 16)] = v

    def compute_chunk(c):
        rbase = wbase + c * CHUNK
        delta = rbase - 2 * (jnp.minimum(rbase, jnp.int32(N - CHUNK)) // 2)

        def init_g(g, _):
            ent = delta + g * 16 + it16
            ridx[g, :] = jnp.minimum(ent, jnp.int32(CHUNK - 1)) * ESTRIDE
            gs = g * 16
            for b in range(B):
                scores[b, pl.ds(gs, 16)] = gamma16
            return 0

        lax.fori_loop(0, GROUPS, init_g, 0)

        def k_body(k, _):
            ksp = jnp.broadcast_to(k, (16,))
            qv = []
            for b in range(B):
                bsp = jnp.full((16,), b + 1, jnp.int32)
                qv.append((plsc.load_gather(qrot, [bsp, ksp]),
                           plsc.load_gather(qrot, [bsp, ksp + HD])))

            @plsc.parallel_loop(0, GROUPS, unroll=2)
            def g_body(g):
                idx_re = ridx[g, :] + ksp
                idx_im = idx_re + HD
                e_re = plsc.load_gather(pbuf, [idx_re])
                e_im = plsc.load_gather(pbuf, [idx_im])
                gs = g * 16
                for b in range(B):
                    qre, qim = qv[b]
                    dre = qre - e_re
                    dim = qim - e_im
                    dsq = dre * dre + dim * dim
                    i = NEG_MAGIC - (plsc.bitcast(dsq, jnp.int32) >> 1)
                    y = plsc.bitcast(i, jnp.float32)
                    y = y * (jnp.float32(1.5)
                             - (jnp.float32(0.5) * dsq) * (y * y))
                    plsc.addupdate(scores.at[b, pl.ds(gs, 16)], dsq * y)

            return 0

        lax.fori_loop(0, HD, k_body, 0)
        col = pl.multiple_of(rbase, 128)
        pltpu.sync_copy(scores, out_hbm.at[:, pl.ds(col, CHUNK)])

    start_load(0)
    for c in range(NCHUNK):
        wait_load()
        relayout()
        if c + 1 < NCHUNK:
            start_load(c + 1)
        compute_chunk(c)


def kernel(all_h, all_r, eemb, remb):
    mesh = plsc.VectorSubcoreMesh(core_axis_name="c", subcore_axis_name="s")
    run = pl.kernel(
        _body,
        out_type=jax.ShapeDtypeStruct((B, NPAD), jnp.float32),
        scratch_types=[
            pltpu.VMEM((B,), jnp.int32),
            pltpu.VMEM((B,), jnp.int32),
            pltpu.VMEM((16,), jnp.int32),
            pltpu.VMEM((B, 128), jnp.float32),
            pltpu.VMEM((B, 128), jnp.float32),
            pltpu.VMEM((B + 1, TD), jnp.float32),
            pltpu.VMEM((GROUPS, 16), jnp.int32),
            pltpu.VMEM((B, CHUNK), jnp.float32),
            pltpu.VMEM((CROWS, 128), jnp.float32),
            pltpu.VMEM((CHUNK * ESTRIDE,), jnp.float32),
            pltpu.SemaphoreType.DMA,
            pltpu.SemaphoreType.DMA,
        ],
        mesh=mesh,
        compiler_params=pltpu.CompilerParams(needs_layout_passes=False),
    )
    all_h = all_h.astype(jnp.int32)
    all_r = all_r.astype(jnp.int32)
    zero4 = jnp.zeros((4,), jnp.int32)
    meta = jnp.concatenate([zero4, (all_h % 2) * TD, (all_r % 4) * HD, zero4])
    out = run(
        all_h // 2, all_r // 4, meta,
        eemb.reshape(N // 2, 128), remb.reshape(125, 128),
    )
    return out[:, :N]


# speedup vs baseline: 1.7756x; 1.0800x over previous
import jax
import jax.numpy as jnp
from jax import lax
from jax.experimental import pallas as pl
from jax.experimental.pallas import tpu as pltpu
from jax.experimental.pallas import tpu_sc as plsc

N = 100000
HD = 32
TD = 64
B = 4
GAMMA = 12.0
PI = 3.141592653589793
EMB_RANGE = (GAMMA + 2.0) / HD
PHASE_SCALE = PI / EMB_RANGE

NW = 32
CHUNK = 640
NCHUNK = 5
PER_W = CHUNK * NCHUNK
NPAD = NW * PER_W
GROUPS = CHUNK // 16
CROWS = CHUNK // 2
ESTRIDE = 65
RPAD = 512

NEG_MAGIC = (0x5F3759DF | 0x80000000) - (1 << 32)


def _sincos16(ph):
    t = ph * jnp.float32(1.0 / PI)
    half = jnp.where(t >= 0, jnp.float32(0.5), jnp.float32(-0.5))
    n = (t + half).astype(jnp.int32)
    y = ph - n.astype(jnp.float32) * jnp.float32(PI)
    y2 = y * y
    cy = jnp.float32(-1.0 / 3628800)
    for c in (1.0 / 40320, -1.0 / 720, 1.0 / 24, -0.5):
        cy = cy * y2 + jnp.float32(c)
    cy = cy * y2 + jnp.float32(1.0)
    sy = jnp.float32(1.0 / 362880)
    for c in (-1.0 / 5040, 1.0 / 120, -1.0 / 6):
        sy = sy * y2 + jnp.float32(c)
    sy = (sy * y2 + jnp.float32(1.0)) * y
    sgn = jnp.where((n & 1) == 0, jnp.float32(1.0), jnp.float32(-1.0))
    return sgn * sy, sgn * cy


def _body(meta_hbm, eemb_hbm, remb_hbm, out_hbm,
          metav, hwin, rwin, qrot, ridx, scores, stage, pbuf, sem0):
    wid = lax.axis_index("s") * 2 + lax.axis_index("c")
    wbase = wid * PER_W
    it16 = lax.iota(jnp.int32, 16)
    gamma16 = jnp.full((16,), jnp.float32(GAMMA))

    def start_half(c, half):
        rbase = wbase + c * CHUNK
        src = pl.multiple_of(
            jnp.minimum(rbase, jnp.int32(N - CHUNK)) + half * CROWS, 8)
        pltpu.make_async_copy(eemb_hbm.at[pl.ds(src, CROWS)], stage, sem0).start()

    def wait_half():
        pltpu.make_async_copy(eemb_hbm.at[pl.ds(0, CROWS)], stage, sem0).wait()

    start_half(0, 0)

    pltpu.sync_copy(meta_hbm, metav)
    for b in range(B):
        bsp = jnp.full((16,), b, jnp.int32)
        hv = plsc.load_gather(metav, [bsp + 4])
        rv = plsc.load_gather(metav, [bsp + 8])
        hs = jnp.max(hv)
        rs = jnp.max(rv)
        hstart = pl.multiple_of((hs // 8) * 8, 8)
        rstart = pl.multiple_of((rs // 8) * 8, 8)
        pltpu.sync_copy(eemb_hbm.at[pl.ds(hstart, 8)], hwin)
        pltpu.sync_copy(remb_hbm.at[pl.ds(rstart, 8)], rwin)
        hsel = hv & 7
        rsel = rv & 7
        for j in range(2):
            lane = it16 + j * 16
            ph = plsc.load_gather(rwin, [rsel, lane])
            ph = ph * jnp.float32(PHASE_SCALE)
            sy, cy = _sincos16(ph)
            reh = plsc.load_gather(hwin, [hsel, lane])
            imh = plsc.load_gather(hwin, [hsel, lane + HD])
            sl = pl.ds(j * 16, 16)
            sli = pl.ds(HD + j * 16, 16)
            qrot[b + 1, sl] = reh * cy - imh * sy
            qrot[b + 1, sli] = reh * sy + imh * cy

    def relayout(half):
        @plsc.parallel_loop(0, CROWS, unroll=2)
        def _(r):
            base = (r + half * CROWS) * ESTRIDE
            for cb in range(4):
                pbuf[pl.ds(base + cb * 16, 16)] = stage[r, pl.ds(cb * 16, 16)]

    def compute_chunk(c):
        rbase = wbase + c * CHUNK
        delta = rbase - jnp.minimum(rbase, jnp.int32(N - CHUNK))

        def init_g(g, _):
            ent = delta + g * 16 + it16
            ridx[g, :] = jnp.minimum(ent, jnp.int32(CHUNK - 1)) * ESTRIDE
            gs = g * 16
            for b in range(B):
                scores[b, pl.ds(gs, 16)] = gamma16
            return 0

        lax.fori_loop(0, GROUPS, init_g, 0)

        def k_body(k, _):
            ksp = jnp.broadcast_to(k, (16,))
            qv = []
            for b in range(B):
                bsp = jnp.full((16,), b + 1, jnp.int32)
                qv.append((plsc.load_gather(qrot, [bsp, ksp]),
                           plsc.load_gather(qrot, [bsp, ksp + HD])))

            @plsc.parallel_loop(0, GROUPS, unroll=2)
            def g_body(g):
                idx_re = ridx[g, :] + ksp
                idx_im = idx_re + HD
                e_re = plsc.load_gather(pbuf, [idx_re])
                e_im = plsc.load_gather(pbuf, [idx_im])
                gs = g * 16
                for b in range(B):
                    qre, qim = qv[b]
                    dre = qre - e_re
                    dim = qim - e_im
                    dsq = dre * dre + dim * dim
                    i = NEG_MAGIC - (plsc.bitcast(dsq, jnp.int32) >> 1)
                    y = plsc.bitcast(i, jnp.float32)
                    y = y * (jnp.float32(1.5)
                             - (jnp.float32(0.5) * dsq) * (y * y))
                    plsc.addupdate(scores.at[b, pl.ds(gs, 16)], dsq * y)

            return 0

        lax.fori_loop(0, HD, k_body, 0)
        col = pl.multiple_of(rbase, 128)
        pltpu.sync_copy(scores, out_hbm.at[:, pl.ds(col, CHUNK)])

    for c in range(NCHUNK):
        wait_half()
        relayout(0)
        start_half(c, 1)
        wait_half()
        relayout(1)
        if c + 1 < NCHUNK:
            start_half(c + 1, 0)
        compute_chunk(c)


def kernel(all_h, all_r, eemb, remb):
    mesh = plsc.VectorSubcoreMesh(core_axis_name="c", subcore_axis_name="s")
    run = pl.kernel(
        _body,
        out_type=jax.ShapeDtypeStruct((B, NPAD), jnp.float32),
        scratch_types=[
            pltpu.VMEM((16,), jnp.int32),
            pltpu.VMEM((8, TD), jnp.float32),
            pltpu.VMEM((8, HD), jnp.float32),
            pltpu.VMEM((B + 1, TD), jnp.float32),
            pltpu.VMEM((GROUPS, 16), jnp.int32),
            pltpu.VMEM((B, CHUNK), jnp.float32),
            pltpu.VMEM((CROWS, TD), jnp.float32),
            pltpu.VMEM((CHUNK * ESTRIDE,), jnp.float32),
            pltpu.SemaphoreType.DMA,
        ],
        mesh=mesh,
        compiler_params=pltpu.CompilerParams(needs_layout_passes=False),
    )
    all_h = all_h.astype(jnp.int32)
    all_r = all_r.astype(jnp.int32)
    zero4 = jnp.zeros((4,), jnp.int32)
    meta = jnp.concatenate([zero4, all_h, all_r, zero4])
    remb_p = jnp.pad(remb, ((0, RPAD - remb.shape[0]), (0, 0)))
    out = run(meta, eemb, remb_p)
    return out[:, :N]
